# Initial kernel scaffold; baseline (speedup 1.0000x reference)
import jax
import jax.numpy as jnp
from jax.experimental import pallas as pl


def kernel(x, W_m, W_u, W_v, eps):
    raise NotImplementedError("write your pallas kernel here")


# speedup vs baseline: 1.3376x; 1.3376x over previous
import functools

import jax
import jax.numpy as jnp
from jax.experimental import pallas as pl
from jax.experimental.pallas import tpu as pltpu

_B, _N, _M = 8192, 4096, 4096
_BM = 512
_BN = 512


def _body(wm_ref, eps_ref, wu_ref, wv_ref, x_ref, o_ref, wbf_ref):
    @pl.when(pl.program_id(1) == 0)
    def _build_w():
        su = jnp.exp(0.5 * wu_ref[...])
        sv = jnp.exp(0.5 * wv_ref[...])
        w = wm_ref[...] + su * (eps_ref[...] * sv)
        wbf_ref[...] = w.astype(jnp.bfloat16)

    o_ref[...] = jnp.dot(x_ref[...], wbf_ref[...],
                         preferred_element_type=jnp.float32)


@functools.partial(jax.jit, static_argnames=("interpret",))
def kernel(x, W_m, W_u, W_v, eps, interpret=False):
    xb = x.astype(jnp.bfloat16)
    wu2 = W_u.reshape(_N, 1)
    wv2 = W_v.reshape(1, _M)
    grid = (_M // _BN, _B // _BM)
    return pl.pallas_call(
        _body,
        grid=grid,
        in_specs=[
            pl.BlockSpec((_N, _BN), lambda j, i: (0, j)),
            pl.BlockSpec((_N, _BN), lambda j, i: (0, j)),
            pl.BlockSpec((_N, 1), lambda j, i: (0, 0)),
            pl.BlockSpec((1, _BN), lambda j, i: (0, j)),
            pl.BlockSpec((_BM, _N), lambda j, i: (i, 0)),
        ],
        out_specs=pl.BlockSpec((_BM, _BN), lambda j, i: (i, j)),
        out_shape=jax.ShapeDtypeStruct((_B, _M), jnp.float32),
        scratch_shapes=[pltpu.VMEM((_N, _BN), jnp.bfloat16)],
        compiler_params=pltpu.CompilerParams(
            dimension_semantics=("parallel", "arbitrary"),
            vmem_limit_bytes=56 * 1024 * 1024,
        ),
        name="mvg_fused_gemm",
        interpret=interpret,
    )(W_m, eps, wu2, wv2, xb)


# speedup vs baseline: 1.4276x; 1.0673x over previous
import functools

import jax
import jax.numpy as jnp
from jax.experimental import pallas as pl
from jax.experimental.pallas import tpu as pltpu

_B, _N, _M = 8192, 4096, 4096
_BM = 1024
_BN = 512


def _body(wm_ref, eps_ref, wu_ref, wv_ref, x_ref, o_ref, wbf_ref):
    @pl.when(pl.program_id(1) == 0)
    def _build_w():
        su = jnp.exp(0.5 * wu_ref[...])
        sv = jnp.exp(0.5 * wv_ref[...])
        w = wm_ref[...] + su * (eps_ref[...] * sv)
        wbf_ref[...] = w.astype(jnp.bfloat16)

    o_ref[...] = jnp.dot(x_ref[...], wbf_ref[...],
                         preferred_element_type=jnp.float32)


@functools.partial(jax.jit, static_argnames=("interpret",))
def kernel(x, W_m, W_u, W_v, eps, interpret=False):
    xb = x.astype(jnp.bfloat16)
    wu2 = W_u.reshape(_N, 1)
    wv2 = W_v.reshape(1, _M)
    grid = (_M // _BN, _B // _BM)
    return pl.pallas_call(
        _body,
        grid=grid,
        in_specs=[
            pl.BlockSpec((_N, _BN), lambda j, i: (0, j)),
            pl.BlockSpec((_N, _BN), lambda j, i: (0, j)),
            pl.BlockSpec((_N, 1), lambda j, i: (0, 0)),
            pl.BlockSpec((1, _BN), lambda j, i: (0, j)),
            pl.BlockSpec((_BM, _N), lambda j, i: (i, 0)),
        ],
        out_specs=pl.BlockSpec((_BM, _BN), lambda j, i: (i, j)),
        out_shape=jax.ShapeDtypeStruct((_B, _M), jnp.float32),
        scratch_shapes=[pltpu.VMEM((_N, _BN), jnp.bfloat16)],
        compiler_params=pltpu.CompilerParams(
            dimension_semantics=("parallel", "arbitrary"),
            vmem_limit_bytes=59904 * 1024,
        ),
        name="mvg_fused_gemm",
        interpret=interpret,
    )(W_m, eps, wu2, wv2, xb)
